# Initial kernel scaffold; baseline (speedup 1.0000x reference)
import jax
import jax.numpy as jnp
from jax.experimental import pallas as pl


def kernel(utterance, dialog, cpt_emb, W_u, b_u, W_c, b_c, W_e, b_e, W_att, b_att, Wf_u, Wf_c, Wf_o, Wf):
    raise NotImplementedError("write your pallas kernel here")


# speedup vs baseline: 2.7914x; 2.7914x over previous
import jax
import jax.numpy as jnp
from jax.experimental import pallas as pl

_B, _T, _C = 16, 32, 128
_H, _MID, _DH = 128, 64, 128


def _seq2seq_kernel(u_ref, c_ref, cpt_ref,
                    W_u_ref, b_u_ref, W_c_ref, b_c_ref, W_e_ref, b_e_ref,
                    W_att_ref, b_att_ref, Wf_u_ref, Wf_c_ref, Wf_o_ref, Wf_ref,
                    out_ref):
    T, C, H, MID = _T, _C, _H, _MID
    u = u_ref[0]
    c = c_ref[0]
    cpt = cpt_ref[0]

    res_u = jnp.dot(u, W_u_ref[...], preferred_element_type=jnp.float32) + b_u_ref[...]
    res_c = jnp.dot(c, W_c_ref[...], preferred_element_type=jnp.float32) + b_c_ref[...]
    s_uc = jnp.dot(res_u + res_c, W_att_ref[...],
                   preferred_element_type=jnp.float32) + b_att_ref[...]

    cpt2 = cpt.reshape(T * C, H)
    res_e = jnp.dot(cpt2, W_e_ref[...], preferred_element_type=jnp.float32) + b_e_ref[...]
    s_e = jnp.dot(res_e, W_att_ref[...],
                  preferred_element_type=jnp.float32).reshape(T, C)

    scores = s_e + s_uc
    mx = jnp.max(scores, axis=1, keepdims=True)
    ex = jnp.exp(scores - mx)
    dist = ex / jnp.sum(ex, axis=1, keepdims=True)

    o = jnp.sum(dist[:, :, None] * cpt, axis=1)

    res_f = (jnp.dot(u, Wf_u_ref[...], preferred_element_type=jnp.float32)
             + jnp.dot(c, Wf_c_ref[...], preferred_element_type=jnp.float32)
             + jnp.dot(o, Wf_o_ref[...], preferred_element_type=jnp.float32))
    g = jax.nn.sigmoid(jnp.dot(res_f, Wf_ref[...],
                               preferred_element_type=jnp.float32))

    t_idx = jax.lax.broadcasted_iota(jnp.int32, (T, 1), 0)
    g_eff = jnp.where(t_idx == 0, 0.0, g)
    lg = jnp.log(g)
    row = jax.lax.broadcasted_iota(jnp.int32, (T, T), 0)
    col = jax.lax.broadcasted_iota(jnp.int32, (T, T), 1)
    umask = (col > row).astype(jnp.float32)
    m = jnp.exp(jnp.dot(umask, lg, preferred_element_type=jnp.float32))
    w = (1.0 - g_eff) * m

    out_ref[0] = dist * w


def kernel(utterance, dialog, cpt_emb, W_u, b_u, W_c, b_c, W_e, b_e,
           W_att, b_att, Wf_u, Wf_c, Wf_o, Wf):
    B, T, C, H = cpt_emb.shape
    MID = W_u.shape[1]
    DH = dialog.shape[2]

    c_shift = jnp.concatenate(
        [jnp.zeros_like(dialog[:, :1]), dialog[:, :-1]], axis=1)

    b_u2 = b_u.reshape(1, MID)
    b_c2 = b_c.reshape(1, MID)
    b_e2 = b_e.reshape(1, MID)
    b_att2 = b_att.reshape(1, 1)

    full = lambda shape: pl.BlockSpec(shape, lambda b: (0,) * len(shape))

    out = pl.pallas_call(
        _seq2seq_kernel,
        grid=(B,),
        in_specs=[
            pl.BlockSpec((1, T, 2 * H), lambda b: (b, 0, 0)),
            pl.BlockSpec((1, T, DH), lambda b: (b, 0, 0)),
            pl.BlockSpec((1, T, C, H), lambda b: (b, 0, 0, 0)),
            full((2 * H, MID)), full((1, MID)),
            full((DH, MID)), full((1, MID)),
            full((H, MID)), full((1, MID)),
            full((MID, 1)), full((1, 1)),
            full((2 * H, MID)), full((DH, MID)), full((H, MID)),
            full((MID, 1)),
        ],
        out_specs=pl.BlockSpec((1, T, C), lambda b: (b, 0, 0)),
        out_shape=jax.ShapeDtypeStruct((B, T, C), jnp.float32),
    )(utterance, c_shift, cpt_emb,
      W_u, b_u2, W_c, b_c2, W_e, b_e2, W_att, b_att2,
      Wf_u, Wf_c, Wf_o, Wf)

    return out.reshape(B, T * C)


# speedup vs baseline: 4.9748x; 1.7822x over previous
import jax
import jax.numpy as jnp
from jax.experimental import pallas as pl

_B, _T, _C = 16, 32, 128
_H, _MID, _DH = 128, 64, 128


def _seq2seq_kernel(u_ref, c_ref, cpt_ref,
                    W_u_ref, b_u_ref, W_c_ref, b_c_ref, W_e_ref, b_e_ref,
                    W_att_ref, b_att_ref, Wf_u_ref, Wf_c_ref, Wf_o_ref, Wf_ref,
                    out_ref):
    T, C, H, MID = _T, _C, _H, _MID
    u = u_ref[0]
    c = c_ref[0]
    cpt = cpt_ref[0]

    res_u = jnp.dot(u, W_u_ref[...], preferred_element_type=jnp.float32) + b_u_ref[...]
    res_c = jnp.dot(c, W_c_ref[...], preferred_element_type=jnp.float32) + b_c_ref[...]
    s_uc = jnp.dot(res_u + res_c, W_att_ref[...],
                   preferred_element_type=jnp.float32) + b_att_ref[...]

    v_att = jnp.dot(W_e_ref[...], W_att_ref[...],
                    preferred_element_type=jnp.float32)
    wfo_f = jnp.dot(Wf_o_ref[...], Wf_ref[...],
                    preferred_element_type=jnp.float32)
    be_att = jnp.dot(b_e_ref[...], W_att_ref[...],
                     preferred_element_type=jnp.float32)
    v2t = jnp.concatenate([v_att, wfo_f], axis=1).T

    cpt2 = cpt.reshape(T * C, H)
    P = jax.lax.dot_general(v2t, cpt2, (((1,), (1,)), ((), ())),
                            preferred_element_type=jnp.float32)
    s_e = P[0:1, :].reshape(T, C)
    q = P[1:2, :].reshape(T, C)

    scores = s_e + (s_uc + be_att)
    mx = jnp.max(scores, axis=1, keepdims=True)
    ex = jnp.exp(scores - mx)
    dist = ex / jnp.sum(ex, axis=1, keepdims=True)

    res_f_uc = (jnp.dot(u, Wf_u_ref[...], preferred_element_type=jnp.float32)
                + jnp.dot(c, Wf_c_ref[...], preferred_element_type=jnp.float32))
    glogit = (jnp.dot(res_f_uc, Wf_ref[...], preferred_element_type=jnp.float32)
              + jnp.sum(dist * q, axis=1, keepdims=True))
    g = jax.nn.sigmoid(glogit)

    t_idx = jax.lax.broadcasted_iota(jnp.int32, (T, 1), 0)
    g_eff = jnp.where(t_idx == 0, 0.0, g)
    lg = jnp.log(g)
    row = jax.lax.broadcasted_iota(jnp.int32, (T, T), 0)
    col = jax.lax.broadcasted_iota(jnp.int32, (T, T), 1)
    umask = (col > row).astype(jnp.float32)
    m = jnp.exp(jnp.dot(umask, lg, preferred_element_type=jnp.float32))
    w = (1.0 - g_eff) * m

    out_ref[0] = dist * w


def kernel(utterance, dialog, cpt_emb, W_u, b_u, W_c, b_c, W_e, b_e,
           W_att, b_att, Wf_u, Wf_c, Wf_o, Wf):
    B, T, C, H = cpt_emb.shape
    MID = W_u.shape[1]
    DH = dialog.shape[2]

    c_shift = jnp.concatenate(
        [jnp.zeros_like(dialog[:, :1]), dialog[:, :-1]], axis=1)

    b_u2 = b_u.reshape(1, MID)
    b_c2 = b_c.reshape(1, MID)
    b_e2 = b_e.reshape(1, MID)
    b_att2 = b_att.reshape(1, 1)

    full = lambda shape: pl.BlockSpec(shape, lambda b: (0,) * len(shape))

    out = pl.pallas_call(
        _seq2seq_kernel,
        grid=(B,),
        in_specs=[
            pl.BlockSpec((1, T, 2 * H), lambda b: (b, 0, 0)),
            pl.BlockSpec((1, T, DH), lambda b: (b, 0, 0)),
            pl.BlockSpec((1, T, C, H), lambda b: (b, 0, 0, 0)),
            full((2 * H, MID)), full((1, MID)),
            full((DH, MID)), full((1, MID)),
            full((H, MID)), full((1, MID)),
            full((MID, 1)), full((1, 1)),
            full((2 * H, MID)), full((DH, MID)), full((H, MID)),
            full((MID, 1)),
        ],
        out_specs=pl.BlockSpec((1, T, C), lambda b: (b, 0, 0)),
        out_shape=jax.ShapeDtypeStruct((B, T, C), jnp.float32),
    )(utterance, c_shift, cpt_emb,
      W_u, b_u2, W_c, b_c2, W_e, b_e2, W_att, b_att2,
      Wf_u, Wf_c, Wf_o, Wf)

    return out.reshape(B, T * C)


# speedup vs baseline: 6.6552x; 1.3378x over previous
import jax
import jax.numpy as jnp
from jax.experimental import pallas as pl

_B, _T, _C = 16, 32, 128
_H, _MID, _DH = 128, 64, 128
_BB = 4


def _seq2seq_kernel(u_ref, c_ref, cpt_ref,
                    W_u_ref, b_u_ref, W_c_ref, b_c_ref, W_e_ref, b_e_ref,
                    W_att_ref, b_att_ref, Wf_u_ref, Wf_c_ref, Wf_o_ref, Wf_ref,
                    out_ref):
    T, C, H, MID, BB = _T, _C, _H, _MID, _BB
    R = BB * T
    u = u_ref[...].reshape(R, 2 * H)
    c = c_ref[...].reshape(R, _DH)

    res_u = jnp.dot(u, W_u_ref[...], preferred_element_type=jnp.float32) + b_u_ref[...]
    res_c = jnp.dot(c, W_c_ref[...], preferred_element_type=jnp.float32) + b_c_ref[...]
    s_uc = jnp.dot(res_u + res_c, W_att_ref[...],
                   preferred_element_type=jnp.float32) + b_att_ref[...]

    v_att = jnp.dot(W_e_ref[...], W_att_ref[...],
                    preferred_element_type=jnp.float32)
    wfo_f = jnp.dot(Wf_o_ref[...], Wf_ref[...],
                    preferred_element_type=jnp.float32)
    be_att = jnp.dot(b_e_ref[...], W_att_ref[...],
                     preferred_element_type=jnp.float32)
    v2t = jnp.concatenate([v_att, wfo_f], axis=1).T

    cpt2 = cpt_ref[...].reshape(R * C, H)
    P = jax.lax.dot_general(v2t, cpt2, (((1,), (1,)), ((), ())),
                            preferred_element_type=jnp.float32)
    s_e = P[0:1, :].reshape(R, C)
    q = P[1:2, :].reshape(R, C)

    scores = s_e + (s_uc + be_att)
    mx = jnp.max(scores, axis=1, keepdims=True)
    ex = jnp.exp(scores - mx)
    dist = ex / jnp.sum(ex, axis=1, keepdims=True)

    res_f_uc = (jnp.dot(u, Wf_u_ref[...], preferred_element_type=jnp.float32)
                + jnp.dot(c, Wf_c_ref[...], preferred_element_type=jnp.float32))
    glogit = (jnp.dot(res_f_uc, Wf_ref[...], preferred_element_type=jnp.float32)
              + jnp.sum(dist * q, axis=1, keepdims=True))
    g = jax.nn.sigmoid(glogit)

    t_idx = jax.lax.broadcasted_iota(jnp.int32, (R, 1), 0)
    g_eff = jnp.where(t_idx % T == 0, 0.0, g)
    lg = jnp.log(g)
    row = jax.lax.broadcasted_iota(jnp.int32, (R, R), 0)
    col = jax.lax.broadcasted_iota(jnp.int32, (R, R), 1)
    umask = ((col > row) & (col // T == row // T)).astype(jnp.float32)
    m = jnp.exp(jnp.dot(umask, lg, preferred_element_type=jnp.float32))
    w = (1.0 - g_eff) * m

    out_ref[...] = (dist * w).reshape(BB, T, C)


def kernel(utterance, dialog, cpt_emb, W_u, b_u, W_c, b_c, W_e, b_e,
           W_att, b_att, Wf_u, Wf_c, Wf_o, Wf):
    B, T, C, H = cpt_emb.shape
    MID = W_u.shape[1]
    DH = dialog.shape[2]
    BB = _BB

    c_shift = jnp.concatenate(
        [jnp.zeros_like(dialog[:, :1]), dialog[:, :-1]], axis=1)

    b_u2 = b_u.reshape(1, MID)
    b_c2 = b_c.reshape(1, MID)
    b_e2 = b_e.reshape(1, MID)
    b_att2 = b_att.reshape(1, 1)

    full = lambda shape: pl.BlockSpec(shape, lambda b: (0,) * len(shape))

    out = pl.pallas_call(
        _seq2seq_kernel,
        grid=(B // BB,),
        in_specs=[
            pl.BlockSpec((BB, T, 2 * H), lambda b: (b, 0, 0)),
            pl.BlockSpec((BB, T, DH), lambda b: (b, 0, 0)),
            pl.BlockSpec((BB, T, C, H), lambda b: (b, 0, 0, 0)),
            full((2 * H, MID)), full((1, MID)),
            full((DH, MID)), full((1, MID)),
            full((H, MID)), full((1, MID)),
            full((MID, 1)), full((1, 1)),
            full((2 * H, MID)), full((DH, MID)), full((H, MID)),
            full((MID, 1)),
        ],
        out_specs=pl.BlockSpec((BB, T, C), lambda b: (b, 0, 0)),
        out_shape=jax.ShapeDtypeStruct((B, T, C), jnp.float32),
    )(utterance, c_shift, cpt_emb,
      W_u, b_u2, W_c, b_c2, W_e, b_e2, W_att, b_att2,
      Wf_u, Wf_c, Wf_o, Wf)

    return out.reshape(B, T * C)
